# baseline (speedup 1.0000x reference)
import jax
import jax.numpy as jnp
from jax.experimental import pallas as pl
from jax.experimental.pallas import tpu as pltpu


def _make_body(N, d, BM, NBUF, nsteps):
    def rows(i):
        return min(BM, N - i * BM)

    def body(alpha_ref, adj_hbm, x_hbm, x0_hbm, out_hbm,
             xv, xvb, x0v, abufs, obufs, asem, xsem, x0sem, osem):
        a = alpha_ref[0]
        pltpu.make_async_copy(x_hbm, xv, xsem).start()
        pltpu.make_async_copy(x0_hbm, x0v, x0sem).start()
        for j in range(min(NBUF, nsteps)):
            pltpu.make_async_copy(
                adj_hbm.at[pl.ds(j * BM, rows(j)), :],
                abufs.at[j, pl.ds(0, rows(j)), :],
                asem.at[j],
            ).start()
        pltpu.make_async_copy(x_hbm, xv, xsem).wait()
        xvb[...] = xv[...].astype(jnp.bfloat16)
        pltpu.make_async_copy(x0_hbm, x0v, x0sem).wait()

        for i in range(nsteps):
            slot = i % NBUF
            oslot = i % 2
            r = rows(i)
            pltpu.make_async_copy(
                adj_hbm.at[pl.ds(i * BM, r), :],
                abufs.at[slot, pl.ds(0, r), :],
                asem.at[slot],
            ).wait()
            if i >= 2:
                pltpu.make_async_copy(
                    obufs.at[oslot, pl.ds(0, rows(i - 2)), :],
                    out_hbm.at[pl.ds((i - 2) * BM, rows(i - 2)), :],
                    osem.at[oslot],
                ).wait()
            prop = jnp.dot(
                abufs[slot, pl.ds(0, r), :].astype(jnp.bfloat16),
                xvb[...],
                preferred_element_type=jnp.float32,
            )
            obufs[oslot, pl.ds(0, r), :] = (
                a * prop + (1.0 - a) * x0v[pl.ds(i * BM, r), :]
            )
            pltpu.make_async_copy(
                obufs.at[oslot, pl.ds(0, r), :],
                out_hbm.at[pl.ds(i * BM, r), :],
                osem.at[oslot],
            ).start()
            if i + NBUF < nsteps:
                pltpu.make_async_copy(
                    adj_hbm.at[pl.ds((i + NBUF) * BM, rows(i + NBUF)), :],
                    abufs.at[slot, pl.ds(0, rows(i + NBUF)), :],
                    asem.at[slot],
                ).start()

        for i in range(max(0, nsteps - 2), nsteps):
            oslot = i % 2
            pltpu.make_async_copy(
                obufs.at[oslot, pl.ds(0, rows(i)), :],
                out_hbm.at[pl.ds(i * BM, rows(i)), :],
                osem.at[oslot],
            ).wait()

    return body


def kernel(x, adj, x_0, alpha):
    N, d = x.shape
    BM = 256
    NBUF = 4
    nsteps = (N + BM - 1) // BM
    return pl.pallas_call(
        _make_body(N, d, BM, NBUF, nsteps),
        in_specs=[
            pl.BlockSpec(memory_space=pltpu.SMEM),
            pl.BlockSpec(memory_space=pltpu.HBM),
            pl.BlockSpec(memory_space=pltpu.HBM),
            pl.BlockSpec(memory_space=pltpu.HBM),
        ],
        out_specs=pl.BlockSpec(memory_space=pltpu.HBM),
        out_shape=jax.ShapeDtypeStruct((N, d), jnp.float32),
        scratch_shapes=[
            pltpu.VMEM((N, d), jnp.float32),
            pltpu.VMEM((N, d), jnp.bfloat16),
            pltpu.VMEM((N, d), jnp.float32),
            pltpu.VMEM((NBUF, BM, N), jnp.float32),
            pltpu.VMEM((2, BM, d), jnp.float32),
            pltpu.SemaphoreType.DMA((NBUF,)),
            pltpu.SemaphoreType.DMA,
            pltpu.SemaphoreType.DMA,
            pltpu.SemaphoreType.DMA((2,)),
        ],
    )(alpha, adj, x, x_0)


# speedup vs baseline: 1.0080x; 1.0080x over previous
import jax
import jax.numpy as jnp
from jax.experimental import pallas as pl
from jax.experimental.pallas import tpu as pltpu


def _appnp_block(alpha_ref, adj_ref, x_ref, x0_ref, out_ref):
    a = alpha_ref[0]
    prop = jnp.dot(
        adj_ref[...].astype(jnp.bfloat16),
        x_ref[...].astype(jnp.bfloat16),
        preferred_element_type=jnp.float32,
    )
    out_ref[...] = a * prop + (1.0 - a) * x0_ref[...]


def kernel(x, adj, x_0, alpha):
    N, d = x.shape
    BM = 256
    return pl.pallas_call(
        _appnp_block,
        grid=(pl.cdiv(N, BM),),
        in_specs=[
            pl.BlockSpec(memory_space=pltpu.SMEM),
            pl.BlockSpec((BM, N), lambda i: (i, 0)),
            pl.BlockSpec((N, d), lambda i: (0, 0)),
            pl.BlockSpec((BM, d), lambda i: (i, 0)),
        ],
        out_specs=pl.BlockSpec((BM, d), lambda i: (i, 0)),
        out_shape=jax.ShapeDtypeStruct((N, d), jnp.float32),
    )(alpha, adj, x, x_0)
